# baseline (speedup 1.0000x reference)
import functools

import jax
import jax.numpy as jnp
from jax import lax
from jax.experimental import pallas as pl
from jax.experimental.pallas import tpu as pltpu
import jax.experimental.pallas.tpu_sc as plsc

NC = 2
NS = 16
NW = NC * NS
LANES = 16
GSEG = 1024
A_LEN = 4 * GSEG
TAIL = 128
ROW = A_LEN + TAIL
CHUNK = 512
COMP = 16
DEPTH = 4


def _build_pack(n):
    ncht = n // CHUNK
    nch_w = -(-ncht // (NW * DEPTH)) * DEPTH
    ncht_pad = nch_w * NW

    def pack_body(pred_ref, targ_ref, mass_ref, bidx_ref, out_ref):
        q = pl.program_id(0)
        p = pred_ref[...]
        t = targ_ref[...]
        m = mass_ref[0]
        b = bidx_ref[0]
        z = jnp.zeros((2, CHUNK), jnp.float32)
        slab = jnp.concatenate([p.T, t.T, m, b, z], axis=0)
        slab = jnp.where(q < ncht, slab, 0.0)
        out_ref[...] = slab[None]

    clamp = lambda q: (jnp.minimum(q, ncht - 1), 0)
    clamp3 = lambda q: (jnp.minimum(q, ncht - 1), 0, 0)
    return pl.pallas_call(
        pack_body,
        grid=(ncht_pad,),
        in_specs=[
            pl.BlockSpec((CHUNK, 6), clamp),
            pl.BlockSpec((CHUNK, 6), clamp),
            pl.BlockSpec((1, 1, CHUNK), clamp3),
            pl.BlockSpec((1, 1, CHUNK), clamp3),
        ],
        out_specs=pl.BlockSpec((1, COMP, CHUNK), lambda q: (q, 0, 0)),
        out_shape=jax.ShapeDtypeStruct((ncht_pad, COMP, CHUNK), jnp.float32),
    ), ncht_pad


def _build_sc(n, ncht_pad):
    nch = ncht_pad // NW

    mesh = plsc.VectorSubcoreMesh(
        core_axis_name="c", subcore_axis_name="s", num_cores=NC,
        num_subcores=NS)

    @functools.partial(
        pl.kernel,
        out_type=jax.ShapeDtypeStruct((NW, ROW), jnp.float32),
        name="physics_loss_sc",
        mesh=mesh,
        compiler_params=pltpu.CompilerParams(
            needs_layout_passes=False, use_tc_tiling_on_sc=True),
        scratch_types=(
            [pltpu.VMEM((COMP, CHUNK), jnp.float32) for _ in range(DEPTH)]
            + [pltpu.VMEM((A_LEN,), jnp.float32),
               pltpu.VMEM((TAIL,), jnp.float32)]
            + [pltpu.SemaphoreType.DMA for _ in range(DEPTH)]
        ),
    )
    def sc_kernel(cat_hbm, out_hbm, b0, b1, b2, b3, acc_ref, tail_ref,
                  s0, s1, s2, s3):
        bufs = (b0, b1, b2, b3)
        sems = (s0, s1, s2, s3)
        wid = lax.axis_index("s") * NC + lax.axis_index("c")

        zeros = jnp.zeros((LANES,), jnp.float32)

        def zero_body(k, carry):
            acc_ref[pl.ds(k * LANES, LANES)] = zeros
            return carry

        lax.fori_loop(0, A_LEN // LANES, zero_body, 0)

        def src(k):
            return cat_hbm.at[wid * nch + k]

        for s in range(DEPTH):
            pltpu.async_copy(src(s), bufs[s], sems[s])

        def group_body(buf):
            def body(i, carry):
                accp, accv, accm, gmx = carry
                sl = pl.ds(i * LANES, LANES)
                x = [buf[c, sl] for c in range(14)]
                m = x[12]
                g = x[13].astype(jnp.int32)
                d = [x[c] - x[c + 6] for c in range(6)]
                accp = accp + d[0] * d[0] + d[1] * d[1] + d[2] * d[2]
                accv = accv + d[3] * d[3] + d[4] * d[4] + d[5] * d[5]
                accm = accm + m
                gmx = jnp.maximum(gmx, g)
                s4g = 4 * g
                plsc.addupdate_scatter(acc_ref, [s4g], m * d[3])
                plsc.addupdate_scatter(acc_ref, [s4g + 1], m * d[4])
                plsc.addupdate_scatter(acc_ref, [s4g + 2], m * d[5])
                ke = (0.5 * m) * (d[3] * (x[3] + x[9]) + d[4] * (x[4] + x[10])
                                  + d[5] * (x[5] + x[11]))
                plsc.addupdate_scatter(acc_ref, [s4g + 3], ke)
                return (accp, accv, accm, gmx)
            return body

        def outer_body(j, carry):
            for s in range(DEPTH):
                k = DEPTH * j + s
                pltpu.make_async_copy(src(k), bufs[s], sems[s]).wait()
                carry = lax.fori_loop(0, CHUNK // LANES, group_body(bufs[s]),
                                      carry)

                @pl.when(k + DEPTH < nch)
                def _():
                    pltpu.async_copy(src(k + DEPTH), bufs[s], sems[s])
            return carry

        init = (zeros, zeros, zeros, jnp.full((LANES,), -1, jnp.int32))
        accp, accv, accm, gmx = lax.fori_loop(0, nch // DEPTH, outer_body,
                                              init)

        tail_ref[pl.ds(0, LANES)] = accp
        tail_ref[pl.ds(LANES, LANES)] = accv
        tail_ref[pl.ds(2 * LANES, LANES)] = accm
        tail_ref[pl.ds(3 * LANES, LANES)] = gmx.astype(jnp.float32)
        for k in range(4, TAIL // LANES):
            tail_ref[pl.ds(k * LANES, LANES)] = zeros

        pltpu.sync_copy(acc_ref, out_hbm.at[wid, pl.ds(0, A_LEN)])
        pltpu.sync_copy(tail_ref, out_hbm.at[wid, pl.ds(A_LEN, TAIL)])

    return sc_kernel


def _build_tc(n):
    nf = float(n)

    def tc_body(part_ref, out_ref):
        x = part_ref[...]
        srow = jnp.sum(x, axis=0, keepdims=True)
        mrow = jnp.max(x, axis=0, keepdims=True)
        col = lax.broadcasted_iota(jnp.int32, (1, ROW), 1)
        in_a = col < A_LEN
        c4 = col % 4
        sq = srow * srow
        mom_sq = jnp.sum(jnp.where(in_a & (c4 < 3), sq, 0.0))
        ke_sq = jnp.sum(jnp.where(in_a & (c4 == 3), sq, 0.0))

        def tail_sum(slot, row):
            m = (col >= A_LEN + slot * LANES) & (col < A_LEN + (slot + 1) * LANES)
            return jnp.sum(jnp.where(m, row, 0.0))

        pos_sum = tail_sum(0, srow)
        vel_sum = tail_sum(1, srow)
        mass_sum = tail_sum(2, srow)
        gmax_m = (col >= A_LEN + 3 * LANES) & (col < A_LEN + 4 * LANES)
        gmax = jnp.max(jnp.where(gmax_m, mrow, -1.0))

        n_graphs = gmax + 1.0
        mass_scale = mass_sum / nf
        s_eff = jnp.where(mass_scale > 0.0, mass_scale, 1.0)
        inv2 = 1.0 / (s_eff * s_eff)

        pos_loss = pos_sum / (3.0 * nf)
        vel_loss = vel_sum / (3.0 * nf)
        momentum_loss = mom_sq * inv2 / (n_graphs * 3.0)
        energy_loss = ke_sq * inv2 / n_graphs
        total = (pos_loss + vel_loss + 0.1 * energy_loss
                 + 0.1 * momentum_loss)

        lane = lax.broadcasted_iota(jnp.int32, (1, 128), 1)
        out = jnp.where(lane == 0, total,
              jnp.where(lane == 1, pos_loss,
              jnp.where(lane == 2, vel_loss,
              jnp.where(lane == 3, energy_loss,
              jnp.where(lane == 4, momentum_loss, 0.0)))))
        out_ref[...] = out

    return pl.pallas_call(
        tc_body,
        out_shape=jax.ShapeDtypeStruct((1, 128), jnp.float32),
    )


@jax.jit
def kernel(pred, target, masses, batch_index):
    n = pred.shape[0]
    ncht = n // CHUNK
    pack, ncht_pad = _build_pack(n)
    cat3 = pack(pred, target, masses.reshape(ncht, 1, CHUNK),
                batch_index.astype(jnp.float32).reshape(ncht, 1, CHUNK))
    partials = _build_sc(n, ncht_pad)(cat3)
    out = _build_tc(n)(partials)
    return (out[0, 0], out[0, 1], out[0, 2], out[0, 3], out[0, 4])


# speedup vs baseline: 3.1900x; 3.1900x over previous
import functools

import jax
import jax.numpy as jnp
from jax import lax
from jax.experimental import pallas as pl
from jax.experimental.pallas import tpu as pltpu
import jax.experimental.pallas.tpu_sc as plsc

NC = 2
NS = 16
NW = NC * NS
LANES = 16
GSEG = 1024
A_LEN = 4 * GSEG
TAIL = 128
ROW = A_LEN + TAIL
CHUNK = 400
DEPTH = 2


def _build_sc(n):
    np_w = n // NW
    nch = np_w // CHUNK
    assert np_w * NW == n and nch * CHUNK == np_w and nch % DEPTH == 0

    mesh = plsc.VectorSubcoreMesh(
        core_axis_name="c", subcore_axis_name="s", num_cores=NC,
        num_subcores=NS)

    @functools.partial(
        pl.kernel,
        out_type=jax.ShapeDtypeStruct((NW, ROW), jnp.float32),
        name="physics_loss_sc",
        mesh=mesh,
        compiler_params=pltpu.CompilerParams(needs_layout_passes=False),
        scratch_types=(
            [pltpu.VMEM((CHUNK, 14), jnp.float32) for _ in range(DEPTH)]
            + [pltpu.VMEM((A_LEN,), jnp.float32),
               pltpu.VMEM((TAIL,), jnp.float32)]
            + [pltpu.SemaphoreType.DMA for _ in range(DEPTH)]
        ),
    )
    def sc_kernel(cat_hbm, out_hbm, b0, b1, acc_ref, tail_ref, s0, s1):
        bufs = (b0, b1)
        sems = (s0, s1)
        wid = lax.axis_index("s") * NC + lax.axis_index("c")

        zeros = jnp.zeros((LANES,), jnp.float32)

        def zero_body(k, carry):
            acc_ref[pl.ds(k * LANES, LANES)] = zeros
            return carry

        lax.fori_loop(0, A_LEN // LANES, zero_body, 0)

        cat_r = cat_hbm.reshape(nch * NW, CHUNK, 14)

        def src(k):
            return cat_r.at[wid * nch + k]

        for s in range(DEPTH):
            pltpu.async_copy(src(s), bufs[s], sems[s])

        iota = lax.iota(jnp.int32, LANES)
        col_idx = [jnp.full((LANES,), c, jnp.int32) for c in range(14)]

        def group_body(buf):
            def body(i, carry):
                accp, accv, accm, gmx = carry
                rows = i * LANES + iota
                x = [plsc.load_gather(buf, [rows, col_idx[c]])
                     for c in range(14)]
                m = x[12]
                g = x[13].astype(jnp.int32)
                d = [x[c] - x[c + 6] for c in range(6)]
                accp = accp + d[0] * d[0] + d[1] * d[1] + d[2] * d[2]
                accv = accv + d[3] * d[3] + d[4] * d[4] + d[5] * d[5]
                accm = accm + m
                gmx = jnp.maximum(gmx, g)
                s4g = 4 * g
                plsc.addupdate_scatter(acc_ref, [s4g], m * d[3])
                plsc.addupdate_scatter(acc_ref, [s4g + 1], m * d[4])
                plsc.addupdate_scatter(acc_ref, [s4g + 2], m * d[5])
                ke = (0.5 * m) * (d[3] * (x[3] + x[9]) + d[4] * (x[4] + x[10])
                                  + d[5] * (x[5] + x[11]))
                plsc.addupdate_scatter(acc_ref, [s4g + 3], ke)
                return (accp, accv, accm, gmx)
            return body

        def outer_body(j, carry):
            for s in range(DEPTH):
                k = DEPTH * j + s
                pltpu.make_async_copy(src(k), bufs[s], sems[s]).wait()
                carry = lax.fori_loop(0, CHUNK // LANES, group_body(bufs[s]),
                                      carry)

                @pl.when(k + DEPTH < nch)
                def _():
                    pltpu.async_copy(src(k + DEPTH), bufs[s], sems[s])
            return carry

        init = (zeros, zeros, zeros, jnp.full((LANES,), -1, jnp.int32))
        accp, accv, accm, gmx = lax.fori_loop(0, nch // DEPTH, outer_body,
                                              init)

        tail_ref[pl.ds(0, LANES)] = accp
        tail_ref[pl.ds(LANES, LANES)] = accv
        tail_ref[pl.ds(2 * LANES, LANES)] = accm
        tail_ref[pl.ds(3 * LANES, LANES)] = gmx.astype(jnp.float32)
        for k in range(4, TAIL // LANES):
            tail_ref[pl.ds(k * LANES, LANES)] = zeros

        pltpu.sync_copy(acc_ref, out_hbm.at[wid, pl.ds(0, A_LEN)])
        pltpu.sync_copy(tail_ref, out_hbm.at[wid, pl.ds(A_LEN, TAIL)])

    return sc_kernel


def _build_tc(n):
    nf = float(n)

    def tc_body(part_ref, out_ref):
        x = part_ref[...]
        srow = jnp.sum(x, axis=0, keepdims=True)
        mrow = jnp.max(x, axis=0, keepdims=True)
        col = lax.broadcasted_iota(jnp.int32, (1, ROW), 1)
        in_a = col < A_LEN
        c4 = col % 4
        sq = srow * srow
        mom_sq = jnp.sum(jnp.where(in_a & (c4 < 3), sq, 0.0))
        ke_sq = jnp.sum(jnp.where(in_a & (c4 == 3), sq, 0.0))

        def tail_sum(slot, row):
            m = (col >= A_LEN + slot * LANES) & (col < A_LEN + (slot + 1) * LANES)
            return jnp.sum(jnp.where(m, row, 0.0))

        pos_sum = tail_sum(0, srow)
        vel_sum = tail_sum(1, srow)
        mass_sum = tail_sum(2, srow)
        gmax_m = (col >= A_LEN + 3 * LANES) & (col < A_LEN + 4 * LANES)
        gmax = jnp.max(jnp.where(gmax_m, mrow, -1.0))

        n_graphs = gmax + 1.0
        mass_scale = mass_sum / nf
        s_eff = jnp.where(mass_scale > 0.0, mass_scale, 1.0)
        inv2 = 1.0 / (s_eff * s_eff)

        pos_loss = pos_sum / (3.0 * nf)
        vel_loss = vel_sum / (3.0 * nf)
        momentum_loss = mom_sq * inv2 / (n_graphs * 3.0)
        energy_loss = ke_sq * inv2 / n_graphs
        total = (pos_loss + vel_loss + 0.1 * energy_loss
                 + 0.1 * momentum_loss)

        lane = lax.broadcasted_iota(jnp.int32, (1, 128), 1)
        out = jnp.where(lane == 0, total,
              jnp.where(lane == 1, pos_loss,
              jnp.where(lane == 2, vel_loss,
              jnp.where(lane == 3, energy_loss,
              jnp.where(lane == 4, momentum_loss, 0.0)))))
        out_ref[...] = out

    return pl.pallas_call(
        tc_body,
        out_shape=jax.ShapeDtypeStruct((1, 128), jnp.float32),
    )


@jax.jit
def kernel(pred, target, masses, batch_index):
    n = pred.shape[0]
    cat = jnp.concatenate(
        [pred, target, masses[:, None],
         batch_index.astype(jnp.float32)[:, None]], axis=1)
    partials = _build_sc(n)(cat)
    out = _build_tc(n)(partials)
    return (out[0, 0], out[0, 1], out[0, 2], out[0, 3], out[0, 4])


# speedup vs baseline: 3.4388x; 1.0780x over previous
import functools

import jax
import jax.numpy as jnp
from jax import lax
from jax.experimental import pallas as pl
from jax.experimental.pallas import tpu as pltpu
import jax.experimental.pallas.tpu_sc as plsc

NC = 2
NS = 16
NW = NC * NS
LANES = 16
GSEG = 1024
A_LEN = 4 * GSEG
TAIL = 128
ROW = A_LEN + TAIL
CHUNK = 400
DEPTH = 2


def _build_sc(n):
    np_w = n // NW
    nch = np_w // CHUNK
    assert np_w * NW == n and nch * CHUNK == np_w and nch % DEPTH == 0

    mesh = plsc.VectorSubcoreMesh(
        core_axis_name="c", subcore_axis_name="s", num_cores=NC,
        num_subcores=NS)

    @functools.partial(
        pl.kernel,
        out_type=jax.ShapeDtypeStruct((NW, ROW), jnp.float32),
        name="physics_loss_sc",
        mesh=mesh,
        compiler_params=pltpu.CompilerParams(needs_layout_passes=False),
        scratch_types=(
            [pltpu.VMEM((CHUNK, 14), jnp.float32) for _ in range(DEPTH)]
            + [pltpu.VMEM((A_LEN,), jnp.float32),
               pltpu.VMEM((TAIL,), jnp.float32)]
            + [pltpu.SemaphoreType.DMA for _ in range(DEPTH)]
        ),
    )
    def sc_kernel(cat_hbm, out_hbm, b0, b1, acc_ref, tail_ref, s0, s1):
        bufs = (b0, b1)
        sems = (s0, s1)
        wid = lax.axis_index("s") * NC + lax.axis_index("c")

        zeros = jnp.zeros((LANES,), jnp.float32)

        def zero_body(k, carry):
            acc_ref[pl.ds(k * LANES, LANES)] = zeros
            return carry

        lax.fori_loop(0, A_LEN // LANES, zero_body, 0)

        cat_r = cat_hbm.reshape(nch * NW, CHUNK, 14)

        def src(k):
            return cat_r.at[wid * nch + k]

        for s in range(DEPTH):
            pltpu.async_copy(src(s), bufs[s], sems[s])

        iota = lax.iota(jnp.int32, LANES)
        col_idx = [jnp.full((LANES,), c, jnp.int32) for c in range(14)]

        def group_body(buf):
            def body(i, carry):
                accp, accv, accm, gmx = carry
                rows = i * LANES + iota
                x = [plsc.load_gather(buf, [rows, col_idx[c]])
                     for c in range(14)]
                m = x[12]
                g = x[13].astype(jnp.int32)
                d = [x[c] - x[c + 6] for c in range(6)]
                accp = accp + d[0] * d[0] + d[1] * d[1] + d[2] * d[2]
                accv = accv + d[3] * d[3] + d[4] * d[4] + d[5] * d[5]
                accm = accm + m
                gmx = jnp.maximum(gmx, g)
                s4g = 4 * g
                plsc.addupdate_scatter(acc_ref, [s4g], m * d[3])
                plsc.addupdate_scatter(acc_ref, [s4g + 1], m * d[4])
                plsc.addupdate_scatter(acc_ref, [s4g + 2], m * d[5])
                ke = (0.5 * m) * (d[3] * (x[3] + x[9]) + d[4] * (x[4] + x[10])
                                  + d[5] * (x[5] + x[11]))
                plsc.addupdate_scatter(acc_ref, [s4g + 3], ke)
                return (accp, accv, accm, gmx)
            return body

        def outer_body(j, carry):
            for s in range(DEPTH):
                k = DEPTH * j + s
                pltpu.make_async_copy(src(k), bufs[s], sems[s]).wait()
                carry = lax.fori_loop(0, CHUNK // LANES, group_body(bufs[s]),
                                      carry)

                @pl.when(k + DEPTH < nch)
                def _():
                    pltpu.async_copy(src(k + DEPTH), bufs[s], sems[s])
            return carry

        init = (zeros, zeros, zeros, jnp.full((LANES,), -1, jnp.int32))
        accp, accv, accm, gmx = lax.fori_loop(0, nch // DEPTH, outer_body,
                                              init)

        tail_ref[pl.ds(0, LANES)] = accp
        tail_ref[pl.ds(LANES, LANES)] = accv
        tail_ref[pl.ds(2 * LANES, LANES)] = accm
        tail_ref[pl.ds(3 * LANES, LANES)] = gmx.astype(jnp.float32)
        for k in range(4, TAIL // LANES):
            tail_ref[pl.ds(k * LANES, LANES)] = zeros

        pltpu.sync_copy(acc_ref, out_hbm.at[wid, pl.ds(0, A_LEN)])
        pltpu.sync_copy(tail_ref, out_hbm.at[wid, pl.ds(A_LEN, TAIL)])

    return sc_kernel


def _build_tc(n):
    nf = float(n)

    def tc_body(part_ref, out_ref):
        x = part_ref[...]
        srow = jnp.sum(x, axis=0, keepdims=True)
        mrow = jnp.max(x, axis=0, keepdims=True)
        col = lax.broadcasted_iota(jnp.int32, (1, ROW), 1)
        in_a = col < A_LEN
        c4 = col % 4
        sq = srow * srow
        mom_sq = jnp.sum(jnp.where(in_a & (c4 < 3), sq, 0.0))
        ke_sq = jnp.sum(jnp.where(in_a & (c4 == 3), sq, 0.0))

        def tail_sum(slot, row):
            m = (col >= A_LEN + slot * LANES) & (col < A_LEN + (slot + 1) * LANES)
            return jnp.sum(jnp.where(m, row, 0.0))

        pos_sum = tail_sum(0, srow)
        vel_sum = tail_sum(1, srow)
        mass_sum = tail_sum(2, srow)
        gmax_m = (col >= A_LEN + 3 * LANES) & (col < A_LEN + 4 * LANES)
        gmax = jnp.max(jnp.where(gmax_m, mrow, -1.0))

        n_graphs = gmax + 1.0
        mass_scale = mass_sum / nf
        s_eff = jnp.where(mass_scale > 0.0, mass_scale, 1.0)
        inv2 = 1.0 / (s_eff * s_eff)

        pos_loss = pos_sum / (3.0 * nf)
        vel_loss = vel_sum / (3.0 * nf)
        momentum_loss = mom_sq * inv2 / (n_graphs * 3.0)
        energy_loss = ke_sq * inv2 / n_graphs
        total = (pos_loss + vel_loss + 0.1 * energy_loss
                 + 0.1 * momentum_loss)

        lane = lax.broadcasted_iota(jnp.int32, (1, 128), 1)
        out = jnp.where(lane == 0, total,
              jnp.where(lane == 1, pos_loss,
              jnp.where(lane == 2, vel_loss,
              jnp.where(lane == 3, energy_loss,
              jnp.where(lane == 4, momentum_loss, 0.0)))))
        out_ref[...] = out

    return pl.pallas_call(
        tc_body,
        out_shape=jax.ShapeDtypeStruct((1, 128), jnp.float32),
    )


NPART = 2


@jax.jit
def kernel(pred, target, masses, batch_index):
    n = pred.shape[0]
    n_p = n // NPART
    bidx_f = batch_index.astype(jnp.float32)
    sc = _build_sc(n_p)
    parts = []
    for p in range(NPART):
        lo, hi = p * n_p, (p + 1) * n_p
        cat = jnp.concatenate(
            [pred[lo:hi], target[lo:hi], masses[lo:hi, None],
             bidx_f[lo:hi, None]], axis=1)
        parts.append(sc(cat))
    partials = jnp.concatenate(parts, axis=0)
    out = _build_tc(n)(partials)
    return (out[0, 0], out[0, 1], out[0, 2], out[0, 3], out[0, 4])


# speedup vs baseline: 3.5161x; 1.0225x over previous
import functools

import jax
import jax.numpy as jnp
from jax import lax
from jax.experimental import pallas as pl
from jax.experimental.pallas import tpu as pltpu
import jax.experimental.pallas.tpu_sc as plsc

NC = 2
NS = 16
NW = NC * NS
LANES = 16
GSEG = 1024
A_LEN = 4 * GSEG
TAIL = 128
ROW = A_LEN + TAIL
CHUNK = 400
NCOLS = 11
DEPTH = 2


def _build_sc(n):
    np_w = n // NW
    nch = np_w // CHUNK
    assert np_w * NW == n and nch * CHUNK == np_w and nch % DEPTH == 0

    mesh = plsc.VectorSubcoreMesh(
        core_axis_name="c", subcore_axis_name="s", num_cores=NC,
        num_subcores=NS)

    @functools.partial(
        pl.kernel,
        out_type=jax.ShapeDtypeStruct((NW, ROW), jnp.float32),
        name="physics_loss_sc",
        mesh=mesh,
        compiler_params=pltpu.CompilerParams(needs_layout_passes=False),
        scratch_types=(
            [pltpu.VMEM((CHUNK, NCOLS), jnp.float32) for _ in range(DEPTH)]
            + [pltpu.VMEM((A_LEN,), jnp.float32),
               pltpu.VMEM((TAIL,), jnp.float32)]
            + [pltpu.SemaphoreType.DMA for _ in range(DEPTH)]
        ),
    )
    def sc_kernel(cat_hbm, out_hbm, b0, b1, acc_ref, tail_ref, s0, s1):
        bufs = (b0, b1)
        sems = (s0, s1)
        wid = lax.axis_index("s") * NC + lax.axis_index("c")

        zeros = jnp.zeros((LANES,), jnp.float32)

        def zero_body(k, carry):
            acc_ref[pl.ds(k * LANES, LANES)] = zeros
            return carry

        lax.fori_loop(0, A_LEN // LANES, zero_body, 0)

        cat_r = cat_hbm.reshape(nch * NW, CHUNK, NCOLS)

        def src(k):
            return cat_r.at[wid * nch + k]

        for s in range(DEPTH):
            pltpu.async_copy(src(s), bufs[s], sems[s])

        iota = lax.iota(jnp.int32, LANES)
        col_idx = [jnp.full((LANES,), c, jnp.int32) for c in range(NCOLS)]

        def group_body(buf):
            def body(i, carry):
                accp, accv, accm, gmx = carry
                rows = i * LANES + iota
                x = [plsc.load_gather(buf, [rows, col_idx[c]])
                     for c in range(NCOLS)]
                d = x[:6]
                s = x[6:9]
                m = x[9]
                g = plsc.bitcast(x[10], jnp.int32)
                accp = accp + d[0] * d[0] + d[1] * d[1] + d[2] * d[2]
                accv = accv + d[3] * d[3] + d[4] * d[4] + d[5] * d[5]
                accm = accm + m
                gmx = jnp.maximum(gmx, g)
                s4g = 4 * g
                plsc.addupdate_scatter(acc_ref, [s4g], m * d[3])
                plsc.addupdate_scatter(acc_ref, [s4g + 1], m * d[4])
                plsc.addupdate_scatter(acc_ref, [s4g + 2], m * d[5])
                ke = (0.5 * m) * (d[3] * s[0] + d[4] * s[1] + d[5] * s[2])
                plsc.addupdate_scatter(acc_ref, [s4g + 3], ke)
                return (accp, accv, accm, gmx)
            return body

        def outer_body(j, carry):
            for s in range(DEPTH):
                k = DEPTH * j + s
                pltpu.make_async_copy(src(k), bufs[s], sems[s]).wait()
                carry = lax.fori_loop(0, CHUNK // LANES, group_body(bufs[s]),
                                      carry)

                @pl.when(k + DEPTH < nch)
                def _():
                    pltpu.async_copy(src(k + DEPTH), bufs[s], sems[s])
            return carry

        init = (zeros, zeros, zeros, jnp.full((LANES,), -1, jnp.int32))
        accp, accv, accm, gmx = lax.fori_loop(0, nch // DEPTH, outer_body,
                                              init)

        tail_ref[pl.ds(0, LANES)] = accp
        tail_ref[pl.ds(LANES, LANES)] = accv
        tail_ref[pl.ds(2 * LANES, LANES)] = accm
        tail_ref[pl.ds(3 * LANES, LANES)] = gmx.astype(jnp.float32)
        for k in range(4, TAIL // LANES):
            tail_ref[pl.ds(k * LANES, LANES)] = zeros

        pltpu.sync_copy(acc_ref, out_hbm.at[wid, pl.ds(0, A_LEN)])
        pltpu.sync_copy(tail_ref, out_hbm.at[wid, pl.ds(A_LEN, TAIL)])

    return sc_kernel


def _build_tc(n):
    nf = float(n)

    def tc_body(part_ref, out_ref):
        x = part_ref[...]
        srow = jnp.sum(x, axis=0, keepdims=True)
        mrow = jnp.max(x, axis=0, keepdims=True)
        col = lax.broadcasted_iota(jnp.int32, (1, ROW), 1)
        in_a = col < A_LEN
        c4 = col % 4
        sq = srow * srow
        mom_sq = jnp.sum(jnp.where(in_a & (c4 < 3), sq, 0.0))
        ke_sq = jnp.sum(jnp.where(in_a & (c4 == 3), sq, 0.0))

        def tail_sum(slot, row):
            m = (col >= A_LEN + slot * LANES) & (col < A_LEN + (slot + 1) * LANES)
            return jnp.sum(jnp.where(m, row, 0.0))

        pos_sum = tail_sum(0, srow)
        vel_sum = tail_sum(1, srow)
        mass_sum = tail_sum(2, srow)
        gmax_m = (col >= A_LEN + 3 * LANES) & (col < A_LEN + 4 * LANES)
        gmax = jnp.max(jnp.where(gmax_m, mrow, -1.0))

        n_graphs = gmax + 1.0
        mass_scale = mass_sum / nf
        s_eff = jnp.where(mass_scale > 0.0, mass_scale, 1.0)
        inv2 = 1.0 / (s_eff * s_eff)

        pos_loss = pos_sum / (3.0 * nf)
        vel_loss = vel_sum / (3.0 * nf)
        momentum_loss = mom_sq * inv2 / (n_graphs * 3.0)
        energy_loss = ke_sq * inv2 / n_graphs
        total = (pos_loss + vel_loss + 0.1 * energy_loss
                 + 0.1 * momentum_loss)

        lane = lax.broadcasted_iota(jnp.int32, (1, 128), 1)
        out = jnp.where(lane == 0, total,
              jnp.where(lane == 1, pos_loss,
              jnp.where(lane == 2, vel_loss,
              jnp.where(lane == 3, energy_loss,
              jnp.where(lane == 4, momentum_loss, 0.0)))))
        out_ref[...] = out

    return pl.pallas_call(
        tc_body,
        out_shape=jax.ShapeDtypeStruct((1, 128), jnp.float32),
    )


NPART = 2


@jax.jit
def kernel(pred, target, masses, batch_index):
    n = pred.shape[0]
    n_p = n // NPART
    bidx_bits = jax.lax.bitcast_convert_type(batch_index.astype(jnp.int32),
                                             jnp.float32)
    sc = _build_sc(n_p)
    parts = []
    for p in range(NPART):
        lo, hi = p * n_p, (p + 1) * n_p
        cat = jnp.concatenate(
            [pred[lo:hi] - target[lo:hi],
             pred[lo:hi, 3:6] + target[lo:hi, 3:6],
             masses[lo:hi, None], bidx_bits[lo:hi, None]],
            axis=1)
        parts.append(sc(cat))
    partials = jnp.concatenate(parts, axis=0)
    out = _build_tc(n)(partials)
    return (out[0, 0], out[0, 1], out[0, 2], out[0, 3], out[0, 4])
